# Initial kernel scaffold; baseline (speedup 1.0000x reference)
import jax
import jax.numpy as jnp
from jax.experimental import pallas as pl


def kernel(query, key, rel_bias_table):
    raise NotImplementedError("write your pallas kernel here")


# speedup vs baseline: 20.8760x; 20.8760x over previous
import functools

import jax
import jax.numpy as jnp
from jax import lax
from jax.experimental import pallas as pl
from jax.experimental.pallas import tpu as pltpu
from jax.experimental.pallas import tpu_sc as plsc

MAXREL = 32
BAND = 2 * MAXREL + 1
H = 16
S = 2048

NUM_CORES = 2
NUM_SUBCORES = 16
NUM_WORKERS = NUM_CORES * NUM_SUBCORES
ROWS_PER_WORKER = S // NUM_WORKERS

RUN = S - MAXREL - 1
STRIP_ROWS = 2 * S
STRIP_WORDS = STRIP_ROWS * H
ROW_WORDS = S * H


def _build_sc_call():
    mesh = plsc.VectorSubcoreMesh(core_axis_name="c", subcore_axis_name="s")

    @functools.partial(
        pl.kernel,
        mesh=mesh,
        out_type=jax.ShapeDtypeStruct((S, ROW_WORDS), jnp.float32),
        scratch_types=[
            pltpu.VMEM((STRIP_WORDS,), jnp.float32),
            pltpu.VMEM((BAND * H,), jnp.float32),
            pltpu.SemaphoreType.DMA,
        ],
        compiler_params=pltpu.CompilerParams(use_tc_tiling_on_sc=False),
    )
    def bias_kernel(tbl_hbm, out_hbm, strip_v, tbl_v, sem):
        pltpu.sync_copy(tbl_hbm, tbl_v)

        for t in range(BAND):
            strip_v[pl.ds((RUN + t) * H, H)] = tbl_v[pl.ds((BAND - 1 - t) * H, H)]

        v_hi = tbl_v[pl.ds((BAND - 1) * H, H)]
        v_lo = tbl_v[pl.ds(0, H)]

        def fill(n, carry):
            strip_v[pl.ds(n * H, H)] = v_hi
            strip_v[pl.ds((RUN + BAND + n) * H, H)] = v_lo
            return carry

        lax.fori_loop(0, RUN, fill, 0)

        wid = lax.axis_index("s") * NUM_CORES + lax.axis_index("c")
        base = wid * ROWS_PER_WORKER
        chunk = 8
        for c in range(ROWS_PER_WORKER // chunk):
            handles = []
            for r in range(chunk):
                i = base + c * chunk + r
                src = strip_v.at[pl.ds((S - 1 - i) * H, ROW_WORDS)]
                handles.append(pltpu.async_copy(src, out_hbm.at[i], sem))
            for hd in handles:
                hd.wait()

    return bias_kernel


_BIAS_CALL = _build_sc_call()


def kernel(query, key, rel_bias_table):
    del query, key
    flat = _BIAS_CALL(rel_bias_table.reshape(-1))
    return flat.reshape(1, 1, S, S, H)
